# baseline (speedup 1.0000x reference)
import functools

import jax
import jax.numpy as jnp
import numpy as np
from jax.experimental import pallas as pl
from jax.experimental.pallas import tpu as pltpu

_TRANS_WEIGHT = 1.0
_ROT_WEIGHT = 100.0
_PI = np.float32(np.pi)
_TWO_PI = np.float32(2.0 * np.pi)

_B = 16384
_MAX_BB = 2048
_NSLOT = 2
_AHEAD = 1


def _plan_blocks():
    first = (256, 512, 1024) + (2048,) * 7 + (256,)
    mid = (2048,) * 8
    last = (2048,) * 7 + (1024, 512, 256, 256)
    blocks = []
    for ch in range(6):
        sizes = first if ch == 0 else (last if ch == 5 else mid)
        off = 0
        for sz in sizes:
            blocks.append((ch, off, sz))
            off += sz
        assert off == _B
    return tuple(blocks)


_BLOCKS = _plan_blocks()


def _wrap_correction(a):
    c = jnp.where(a > _PI, jnp.float32(-_TWO_PI), jnp.float32(0.0))
    return jnp.where(a < -_PI, jnp.float32(_TWO_PI), c)


def _loss_kernel(p_hbm, t_hbm, out_ref, pbuf, tbuf, psem, tsem, *, inv_n):
    def dma(hbm, buf, sem, j, slot):
        ch, off, sz = _BLOCKS[j]
        return pltpu.make_async_copy(
            hbm.at[ch, pl.ds(off, sz), :], buf.at[slot, 0:sz, :],
            sem.at[slot])

    for k in range(min(_AHEAD, len(_BLOCKS))):
        dma(p_hbm, pbuf, psem, k, k % _NSLOT).start()
        dma(t_hbm, tbuf, tsem, k, k % _NSLOT).start()

    acc_t = jnp.zeros((8, 128), jnp.float32)
    acc_r = jnp.zeros((8, 128), jnp.float32)
    for j, (ch, off, sz) in enumerate(_BLOCKS):
        slot = j % _NSLOT
        if j + _AHEAD < len(_BLOCKS):
            nxt = (j + _AHEAD) % _NSLOT
            dma(p_hbm, pbuf, psem, j + _AHEAD, nxt).start()
            dma(t_hbm, tbuf, tsem, j + _AHEAD, nxt).start()
        dma(p_hbm, pbuf, psem, j, slot).wait()
        dma(t_hbm, tbuf, tsem, j, slot).wait()
        for i in range(sz // 8):
            p = pbuf[slot, i * 8:(i + 1) * 8, :]
            t = tbuf[slot, i * 8:(i + 1) * 8, :]
            if ch < 3:
                d = p - t
                acc_t = acc_t + d * d
            else:
                d = (p - t) + (_wrap_correction(p) - _wrap_correction(t))
                acc_r = acc_r + d * d

    trans_loss = jnp.sum(acc_t) * inv_n * _TRANS_WEIGHT
    rot_loss = jnp.sum(acc_r) * inv_n * _ROT_WEIGHT
    out_ref[0, 0] = trans_loss + rot_loss
    out_ref[0, 1] = trans_loss
    out_ref[0, 2] = rot_loss


def kernel(pred, target, *, interpret=False):
    B, T, D = pred.shape
    p3 = jnp.transpose(pred, (2, 0, 1))
    t3 = jnp.transpose(target, (2, 0, 1))

    n_per_half = B * T * 3
    out = pl.pallas_call(
        functools.partial(_loss_kernel, inv_n=np.float32(1.0 / n_per_half)),
        in_specs=[
            pl.BlockSpec(memory_space=pl.ANY),
            pl.BlockSpec(memory_space=pl.ANY),
        ],
        out_specs=pl.BlockSpec(memory_space=pltpu.SMEM),
        out_shape=jax.ShapeDtypeStruct((1, 3), jnp.float32),
        scratch_shapes=[
            pltpu.VMEM((_NSLOT, _MAX_BB, T), jnp.float32),
            pltpu.VMEM((_NSLOT, _MAX_BB, T), jnp.float32),
            pltpu.SemaphoreType.DMA((_NSLOT,)),
            pltpu.SemaphoreType.DMA((_NSLOT,)),
        ],
        name="weighted_mse_loss",
        interpret=interpret,
    )(p3, t3)

    return (out[0, 0], out[0, 1], out[0, 2])


# speedup vs baseline: 1.5682x; 1.5682x over previous
import functools

import jax
import jax.numpy as jnp
import numpy as np
from jax.experimental import pallas as pl
from jax.experimental.pallas import tpu as pltpu

_TRANS_WEIGHT = 1.0
_ROT_WEIGHT = 100.0
_PI = np.float32(np.pi)
_TWO_PI = np.float32(2.0 * np.pi)


def _wrap_correction(a):
    c = jnp.where(a > _PI, jnp.float32(-_TWO_PI), jnp.float32(0.0))
    return jnp.where(a < -_PI, jnp.float32(_TWO_PI), c)


def _loss_kernel(p_ref, t_ref, out_ref, acc_ref, *, n_steps, bbr, inv_n):
    j = pl.program_id(0)

    acc_t = jnp.zeros((8, 128), jnp.float32)
    acc_r = jnp.zeros((8, 128), jnp.float32)
    for ch in range(6):
        for i in range(bbr // 8):
            p = p_ref[ch, i * 8:(i + 1) * 8, :]
            t = t_ref[ch, i * 8:(i + 1) * 8, :]
            if ch < 3:
                d = p - t
                acc_t = acc_t + d * d
            else:
                d = (p - t) + (_wrap_correction(p) - _wrap_correction(t))
                acc_r = acc_r + d * d

    @pl.when(j == 0)
    def _():
        acc_ref[0:8, :] = acc_t
        acc_ref[8:16, :] = acc_r

    @pl.when(j > 0)
    def _():
        acc_ref[0:8, :] += acc_t
        acc_ref[8:16, :] += acc_r

    @pl.when(j == n_steps - 1)
    def _():
        s_trans = jnp.sum(acc_ref[0:8, :])
        s_rot = jnp.sum(acc_ref[8:16, :])
        trans_loss = s_trans * inv_n * _TRANS_WEIGHT
        rot_loss = s_rot * inv_n * _ROT_WEIGHT
        out_ref[0, 0] = trans_loss + rot_loss
        out_ref[0, 1] = trans_loss
        out_ref[0, 2] = rot_loss


def kernel(pred, target, *, interpret=False):
    B, T, D = pred.shape
    p3 = jnp.transpose(pred, (2, 0, 1))
    t3 = jnp.transpose(target, (2, 0, 1))

    BBR = 2048
    G = B // BBR
    n_per_half = B * T * 3
    out = pl.pallas_call(
        functools.partial(
            _loss_kernel, n_steps=G, bbr=BBR, inv_n=np.float32(1.0 / n_per_half)
        ),
        grid=(G,),
        in_specs=[
            pl.BlockSpec((D, BBR, T), lambda j: (0, j, 0)),
            pl.BlockSpec((D, BBR, T), lambda j: (0, j, 0)),
        ],
        out_specs=pl.BlockSpec(memory_space=pltpu.SMEM),
        out_shape=jax.ShapeDtypeStruct((1, 3), jnp.float32),
        scratch_shapes=[pltpu.VMEM((16, 128), jnp.float32)],
        compiler_params=pltpu.CompilerParams(
            dimension_semantics=("arbitrary",),
        ),
        name="weighted_mse_loss",
        interpret=interpret,
    )(p3, t3)

    return (out[0, 0], out[0, 1], out[0, 2])
